# baseline (speedup 1.0000x reference)
import functools
import math

import jax
import jax.numpy as jnp
from jax import lax
from jax.experimental import pallas as pl
from jax.experimental.pallas import tpu as pltpu
from jax.experimental.pallas import tpu_sc as plsc

DIM = 64
ROW_W = 128
LANES = 16

_NC = 2
_NS = 16
_NW = _NC * _NS

_CHUNK = 256
_IDX_W = 128


def _rsqrt_scalar(x):
    i = lax.bitcast_convert_type(x, jnp.int32)
    i = jnp.int32(0x5F3759DF) - lax.shift_right_logical(i, 1)
    r = lax.bitcast_convert_type(i, jnp.float32)
    hx = 0.5 * x
    for _ in range(3):
        r = r * (1.5 - hx * r * r)
    return r


def _fused_embed_ln(table128, idx, pe, gamma, beta, n_rows):
    per_w = n_rows // _NW
    n_chunks = per_w // _CHUNK
    n_streams = _CHUNK // _IDX_W
    seq_len = pe.shape[0]

    mesh = plsc.VectorSubcoreMesh(
        core_axis_name="c", subcore_axis_name="s",
        num_cores=_NC, num_subcores=_NS)

    @functools.partial(
        pl.kernel,
        mesh=mesh,
        out_type=jax.ShapeDtypeStruct((n_rows, DIM), jnp.float32),
        scratch_types=[
            pltpu.VMEM((2, _CHUNK), jnp.int32),
            pltpu.VMEM((2, _CHUNK, ROW_W), jnp.float32),
            pltpu.VMEM((2, _CHUNK, DIM), jnp.float32),
            pltpu.VMEM((seq_len, DIM), jnp.float32),
            pltpu.VMEM((DIM,), jnp.float32),
            pltpu.VMEM((DIM,), jnp.float32),
            pltpu.SemaphoreType.DMA,
            pltpu.SemaphoreType.DMA,
        ],
        compiler_params=pltpu.CompilerParams(
            use_tc_tiling_on_sc=False, needs_layout_passes=False),
    )
    def k(table_hbm, idx_hbm, pe_hbm, g_hbm, b_hbm, out_hbm,
          idx_v, rows_v, stage_v, pe_v, g_v, b_v, sem_g, sem_o):
        wid = lax.axis_index("s") * _NC + lax.axis_index("c")
        wbase = wid * per_w

        pltpu.sync_copy(pe_hbm, pe_v)
        pltpu.sync_copy(g_hbm, g_v)
        pltpu.sync_copy(b_hbm, b_v)

        g_regs = [g_v[pl.ds(LANES * t, LANES)] for t in range(DIM // LANES)]
        b_regs = [b_v[pl.ds(LANES * t, LANES)] for t in range(DIM // LANES)]

        def fire_gather(c):
            p = c % 2
            base = wbase + c * _CHUNK
            pltpu.sync_copy(idx_hbm.at[pl.ds(base, _CHUNK)], idx_v.at[p])
            return [
                pltpu.async_copy(
                    table_hbm.at[idx_v.at[p, pl.ds(j * _IDX_W, _IDX_W)]],
                    rows_v.at[p, pl.ds(j * _IDX_W, _IDX_W)],
                    sem_g)
                for j in range(n_streams)
            ]

        def compute(c):
            p = c % 2
            base = wbase + c * _CHUNK

            def row_body(r, _):
                l = lax.rem(base + r, seq_len)
                e = [rows_v[p, r, pl.ds(LANES * t, LANES)]
                     + pe_v[l, pl.ds(LANES * t, LANES)]
                     for t in range(DIM // LANES)]
                s = (e[0] + e[1]) + (e[2] + e[3])
                q = (e[0] * e[0] + e[1] * e[1]) + (e[2] * e[2] + e[3] * e[3])
                tot = jnp.sum(s)
                tot2 = jnp.sum(q)
                mu = tot * (1.0 / DIM)
                var = tot2 * (1.0 / DIM) - mu * mu
                rstd = _rsqrt_scalar(var + 1e-5)
                for t in range(DIM // LANES):
                    stage_v[p, r, pl.ds(LANES * t, LANES)] = (
                        (e[t] - mu) * rstd * g_regs[t] + b_regs[t])
                return ()

            lax.fori_loop(0, _CHUNK, row_body, (), unroll=2)
            return pltpu.async_copy(
                stage_v.at[p], out_hbm.at[pl.ds(base, _CHUNK)], sem_o)

        gather_descs = fire_gather(0)
        out_descs = [None, None]
        for c in range(n_chunks):
            nxt = fire_gather(c + 1) if c + 1 < n_chunks else None
            for d in gather_descs:
                d.wait()
            if out_descs[c % 2] is not None:
                out_descs[c % 2].wait()
            out_descs[c % 2] = compute(c)
            gather_descs = nxt
        for d in out_descs:
            if d is not None:
                d.wait()

    return k(table128, idx, pe, gamma, beta)


def _pe_table(length, d):
    position = jnp.arange(length, dtype=jnp.float32)[:, None]
    div_term = jnp.exp(
        jnp.arange(0, d, 2, dtype=jnp.float32) * (-math.log(10000.0) / d))
    ang = position * div_term
    return jnp.stack([jnp.sin(ang), jnp.cos(ang)], axis=-1).reshape(length, d)


def kernel(x, word_embeddings_weight, ln_gamma, ln_beta):
    b, l = x.shape
    n = b * l
    table128 = jnp.pad(word_embeddings_weight, ((0, 0), (0, ROW_W - DIM)))
    pe = _pe_table(l, DIM)
    out = _fused_embed_ln(table128, x.reshape(n), pe, ln_gamma, ln_beta, n)
    return out.reshape(b, l, DIM)
